# baseline (speedup 1.0000x reference)
import functools

import jax
import jax.numpy as jnp
from jax import lax
from jax.experimental import pallas as pl
from jax.experimental.pallas import tpu as pltpu
from jax.experimental.pallas import tpu_sc as plsc

N = 10000
F = 128
E = 320000

NPAD = 10240
NC = 2
NS = 16
NW = NC * NS
CHUNK = 128
EPAD = 327680
EPW = EPAD // NW
NCHUNK = EPW // CHUNK
ROWS_PER_TILE = NPAD // NS

_mesh = plsc.VectorSubcoreMesh(core_axis_name="c", subcore_axis_name="s")


@functools.partial(
    pl.kernel,
    out_type=jax.ShapeDtypeStruct((NC, NPAD, 1), jnp.float32),
    mesh=_mesh,
    scratch_types=[
        pltpu.VMEM((NCHUNK, CHUNK), jnp.int32),
        pltpu.VMEM((CHUNK, 1), jnp.float32),
        pltpu.VMEM_SHARED((NPAD, 1), jnp.float32),
    ],
)
def _sc_degree(dst_hbm, zeros_hbm, ones_hbm, out_hbm, idx_v, ones_v, deg_sh):
    cid = lax.axis_index("c")
    sid = lax.axis_index("s")
    wid = sid * NC + cid
    row0 = sid * ROWS_PER_TILE
    pltpu.sync_copy(dst_hbm.at[wid], idx_v)
    pltpu.sync_copy(zeros_hbm.at[pl.ds(row0, ROWS_PER_TILE)],
                    deg_sh.at[pl.ds(row0, ROWS_PER_TILE)])
    pltpu.sync_copy(ones_hbm, ones_v)
    plsc.subcore_barrier()

    def body(j, carry):
        pltpu.sync_copy(ones_v, deg_sh.at[idx_v.at[j]], add=True)
        return carry

    lax.fori_loop(0, NCHUNK, body, 0)
    plsc.subcore_barrier()
    pltpu.sync_copy(
        deg_sh.at[pl.ds(row0, ROWS_PER_TILE)],
        out_hbm.at[cid, pl.ds(row0, ROWS_PER_TILE)],
    )


@functools.partial(
    pl.kernel,
    out_type=jax.ShapeDtypeStruct((NC, NPAD, F), jnp.float32),
    mesh=_mesh,
    scratch_types=[
        pltpu.VMEM((NCHUNK, CHUNK), jnp.int32),
        pltpu.VMEM((2, CHUNK), jnp.int32),
        pltpu.VMEM((2, CHUNK, F), jnp.float32),
        pltpu.VMEM_SHARED((NPAD, F), jnp.float32),
        pltpu.SemaphoreType.DMA((2,)),
        pltpu.SemaphoreType.DMA((2,)),
    ],
)
def _sc_scatter(src_hbm, dst_hbm, z_hbm, zrows_hbm, out_hbm,
                src_v, dst_v, rows_v, y_sh, sems, semd):
    cid = lax.axis_index("c")
    sid = lax.axis_index("s")
    wid = sid * NC + cid
    row0 = sid * ROWS_PER_TILE
    pltpu.sync_copy(src_hbm.at[wid], src_v)
    pltpu.sync_copy(zrows_hbm.at[pl.ds(row0, ROWS_PER_TILE)],
                    y_sh.at[pl.ds(row0, ROWS_PER_TILE)])
    plsc.subcore_barrier()
    pltpu.async_copy(z_hbm.at[src_v.at[0]], rows_v.at[0], sems.at[0])
    pltpu.async_copy(dst_hbm.at[wid, 0], dst_v.at[0], semd.at[0])

    def body(j, carry):
        slot = lax.rem(j, 2)
        nslot = lax.rem(j + 1, 2)

        @pl.when(j + 1 < NCHUNK)
        def _start_next():
            pltpu.async_copy(
                z_hbm.at[src_v.at[j + 1]], rows_v.at[nslot], sems.at[nslot])
            pltpu.async_copy(
                dst_hbm.at[wid, j + 1], dst_v.at[nslot], semd.at[nslot])

        pltpu.make_async_copy(
            z_hbm.at[src_v.at[j]], rows_v.at[slot], sems.at[slot]).wait()
        pltpu.make_async_copy(
            dst_hbm.at[wid, j], dst_v.at[slot], semd.at[slot]).wait()
        pltpu.sync_copy(rows_v.at[slot], y_sh.at[dst_v.at[slot]], add=True)
        return carry

    lax.fori_loop(0, NCHUNK, body, 0)
    plsc.subcore_barrier()
    pltpu.sync_copy(
        y_sh.at[pl.ds(row0, ROWS_PER_TILE)],
        out_hbm.at[cid, pl.ds(row0, ROWS_PER_TILE)],
    )


BM = 2048


def _tc1_body(x_ref, deg_ref, w_ref, z_ref, dis_ref):
    d = deg_ref[...]
    deg = d[0] + d[1] + 1.0
    dis = lax.rsqrt(deg)
    xw = jnp.dot(x_ref[...], w_ref[...], preferred_element_type=jnp.float32)
    z_ref[...] = xw * dis
    dis_ref[...] = dis


def _tc1(x, deg, w1):
    return pl.pallas_call(
        _tc1_body,
        grid=(NPAD // BM,),
        in_specs=[
            pl.BlockSpec((BM, F), lambda i: (i, 0)),
            pl.BlockSpec((NC, BM, 1), lambda i: (0, i, 0)),
            pl.BlockSpec((F, F), lambda i: (0, 0)),
        ],
        out_specs=[
            pl.BlockSpec((BM, F), lambda i: (i, 0)),
            pl.BlockSpec((BM, 1), lambda i: (i, 0)),
        ],
        out_shape=[
            jax.ShapeDtypeStruct((NPAD, F), jnp.float32),
            jax.ShapeDtypeStruct((NPAD, 1), jnp.float32),
        ],
    )(x, deg, w1)


def _tc2_body(y0_ref, y1_ref, z1_ref, dis_ref, b1_ref, w2_ref, z2_ref):
    dis = dis_ref[...]
    h = dis * (y0_ref[...] + y1_ref[...] + z1_ref[...]) + b1_ref[...]
    h = jnp.maximum(h, 0.0)
    z2_ref[...] = jnp.dot(h, w2_ref[...], preferred_element_type=jnp.float32) * dis


def _tc2(y0, y1, z1, dis, b1, w2):
    return pl.pallas_call(
        _tc2_body,
        grid=(NPAD // BM,),
        in_specs=[
            pl.BlockSpec((BM, F), lambda i: (i, 0)),
            pl.BlockSpec((BM, F), lambda i: (i, 0)),
            pl.BlockSpec((BM, F), lambda i: (i, 0)),
            pl.BlockSpec((BM, 1), lambda i: (i, 0)),
            pl.BlockSpec((1, F), lambda i: (0, 0)),
            pl.BlockSpec((F, F), lambda i: (0, 0)),
        ],
        out_specs=pl.BlockSpec((BM, F), lambda i: (i, 0)),
        out_shape=jax.ShapeDtypeStruct((NPAD, F), jnp.float32),
    )(y0, y1, z1, dis, b1, w2)


def _tc3_body(y0_ref, y1_ref, z2_ref, dis_ref, b2_ref, out_ref):
    o = dis_ref[...] * (y0_ref[...] + y1_ref[...] + z2_ref[...]) + b2_ref[...]
    out_ref[...] = jnp.maximum(o, 0.0)


BM3 = 2000


def _tc3(y0, y1, z2, dis, b2):
    return pl.pallas_call(
        _tc3_body,
        grid=(N // BM3,),
        in_specs=[
            pl.BlockSpec((BM3, F), lambda i: (i, 0)),
            pl.BlockSpec((BM3, F), lambda i: (i, 0)),
            pl.BlockSpec((BM3, F), lambda i: (i, 0)),
            pl.BlockSpec((BM3, 1), lambda i: (i, 0)),
            pl.BlockSpec((1, F), lambda i: (0, 0)),
        ],
        out_specs=pl.BlockSpec((BM3, F), lambda i: (i, 0)),
        out_shape=jax.ShapeDtypeStruct((N, F), jnp.float32),
    )(y0, y1, z2, dis, b2)


@jax.jit
def kernel(x, edge_index, W1, b1, W2, b2):
    src = edge_index[0].astype(jnp.int32)
    dst = edge_index[1].astype(jnp.int32)
    pad_src = jnp.arange(EPAD - E, dtype=jnp.int32) % N
    pad_dst = jnp.full((EPAD - E,), N, dtype=jnp.int32)
    src_p = jnp.concatenate([src, pad_src]).reshape(NW, NCHUNK, CHUNK)
    dst_p = jnp.concatenate([dst, pad_dst]).reshape(NW, NCHUNK, CHUNK)

    zeros_n = jnp.zeros((NPAD, 1), jnp.float32)
    ones_chunk = jnp.ones((CHUNK, 1), jnp.float32)
    zeros_rows = jnp.zeros((NPAD, F), jnp.float32)

    deg = _sc_degree(dst_p, zeros_n, ones_chunk)

    z1, dis = _tc1(x, deg, W1)
    y1 = _sc_scatter(src_p, dst_p, z1, zeros_rows)
    z2 = _tc2(y1[0], y1[1], z1, dis, b1.reshape(1, F), W2)
    y2 = _sc_scatter(src_p, dst_p, z2, zeros_rows)
    return _tc3(y2[0], y2[1], z2, dis, b2.reshape(1, F))


# speedup vs baseline: 1.0046x; 1.0046x over previous
import functools

import jax
import jax.numpy as jnp
from jax import lax
from jax.experimental import pallas as pl
from jax.experimental.pallas import tpu as pltpu
from jax.experimental.pallas import tpu_sc as plsc

N = 10000
F = 128
E = 320000

NPAD = 10240
NC = 2
NS = 16
NW = NC * NS
CHUNK = 128
EPAD = 327680
EPW = EPAD // NW
NCHUNK = EPW // CHUNK
ROWS_PER_TILE = NPAD // NS

_mesh = plsc.VectorSubcoreMesh(core_axis_name="c", subcore_axis_name="s")


@functools.partial(
    pl.kernel,
    out_type=jax.ShapeDtypeStruct((NC, NPAD), jnp.float32),
    mesh=_mesh,
    scratch_types=[
        pltpu.VMEM((NCHUNK, CHUNK), jnp.int32),
        pltpu.VMEM((CHUNK,), jnp.float32),
        pltpu.VMEM_SHARED((NPAD,), jnp.float32),
    ],
)
def _sc_degree(dst_hbm, zeros_hbm, ones_hbm, out_hbm, idx_v, ones_v, deg_sh):
    cid = lax.axis_index("c")
    sid = lax.axis_index("s")
    wid = sid * NC + cid
    row0 = sid * ROWS_PER_TILE
    pltpu.sync_copy(dst_hbm.at[wid], idx_v)
    pltpu.sync_copy(zeros_hbm.at[pl.ds(row0, ROWS_PER_TILE)],
                    deg_sh.at[pl.ds(row0, ROWS_PER_TILE)])
    pltpu.sync_copy(ones_hbm, ones_v)
    plsc.subcore_barrier()

    def body(j, carry):
        pltpu.sync_copy(ones_v, deg_sh.at[idx_v.at[j]], add=True)
        return carry

    lax.fori_loop(0, NCHUNK, body, 0)
    plsc.subcore_barrier()
    pltpu.sync_copy(
        deg_sh.at[pl.ds(row0, ROWS_PER_TILE)],
        out_hbm.at[cid, pl.ds(row0, ROWS_PER_TILE)],
    )


@functools.partial(
    pl.kernel,
    out_type=jax.ShapeDtypeStruct((NC, NPAD, F), jnp.float32),
    mesh=_mesh,
    scratch_types=[
        pltpu.VMEM((NCHUNK, CHUNK), jnp.int32),
        pltpu.VMEM((2, CHUNK), jnp.int32),
        pltpu.VMEM((2, CHUNK, F), jnp.float32),
        pltpu.VMEM_SHARED((NPAD, F), jnp.float32),
        pltpu.SemaphoreType.DMA((2,)),
        pltpu.SemaphoreType.DMA((2,)),
    ],
)
def _sc_scatter(src_hbm, dst_hbm, z_hbm, zrows_hbm, out_hbm,
                src_v, dst_v, rows_v, y_sh, sems, semd):
    cid = lax.axis_index("c")
    sid = lax.axis_index("s")
    wid = sid * NC + cid
    row0 = sid * ROWS_PER_TILE
    pltpu.sync_copy(src_hbm.at[wid], src_v)
    pltpu.sync_copy(zrows_hbm.at[pl.ds(row0, ROWS_PER_TILE)],
                    y_sh.at[pl.ds(row0, ROWS_PER_TILE)])
    plsc.subcore_barrier()
    pltpu.async_copy(z_hbm.at[src_v.at[0]], rows_v.at[0], sems.at[0])
    pltpu.async_copy(dst_hbm.at[wid, 0], dst_v.at[0], semd.at[0])

    def body(j, carry):
        slot = lax.rem(j, 2)
        nslot = lax.rem(j + 1, 2)

        @pl.when(j + 1 < NCHUNK)
        def _start_next():
            pltpu.async_copy(
                z_hbm.at[src_v.at[j + 1]], rows_v.at[nslot], sems.at[nslot])
            pltpu.async_copy(
                dst_hbm.at[wid, j + 1], dst_v.at[nslot], semd.at[nslot])

        pltpu.make_async_copy(
            z_hbm.at[src_v.at[j]], rows_v.at[slot], sems.at[slot]).wait()
        pltpu.make_async_copy(
            dst_hbm.at[wid, j], dst_v.at[slot], semd.at[slot]).wait()
        pltpu.sync_copy(rows_v.at[slot], y_sh.at[dst_v.at[slot]], add=True)
        return carry

    lax.fori_loop(0, NCHUNK, body, 0)
    plsc.subcore_barrier()
    pltpu.sync_copy(
        y_sh.at[pl.ds(row0, ROWS_PER_TILE)],
        out_hbm.at[cid, pl.ds(row0, ROWS_PER_TILE)],
    )


BM = 2048


def _tc1_body(x_ref, d0_ref, d1_ref, w_ref, z_ref, dis_ref):
    deg = d0_ref[...] + d1_ref[...] + 1.0
    dis = lax.rsqrt(deg)
    xw = jnp.dot(x_ref[...], w_ref[...], preferred_element_type=jnp.float32)
    z_ref[...] = xw * dis
    dis_ref[...] = dis


def _tc1(x, d0, d1, w1):
    return pl.pallas_call(
        _tc1_body,
        grid=(NPAD // BM,),
        in_specs=[
            pl.BlockSpec((BM, F), lambda i: (i, 0)),
            pl.BlockSpec((BM, 1), lambda i: (i, 0)),
            pl.BlockSpec((BM, 1), lambda i: (i, 0)),
            pl.BlockSpec((F, F), lambda i: (0, 0)),
        ],
        out_specs=[
            pl.BlockSpec((BM, F), lambda i: (i, 0)),
            pl.BlockSpec((BM, 1), lambda i: (i, 0)),
        ],
        out_shape=[
            jax.ShapeDtypeStruct((NPAD, F), jnp.float32),
            jax.ShapeDtypeStruct((NPAD, 1), jnp.float32),
        ],
    )(x, d0, d1, w1)


def _tc2_body(y0_ref, y1_ref, z1_ref, dis_ref, b1_ref, w2_ref, z2_ref):
    dis = dis_ref[...]
    h = dis * (y0_ref[...] + y1_ref[...] + z1_ref[...]) + b1_ref[...]
    h = jnp.maximum(h, 0.0)
    z2_ref[...] = jnp.dot(h, w2_ref[...], preferred_element_type=jnp.float32) * dis


def _tc2(y0, y1, z1, dis, b1, w2):
    return pl.pallas_call(
        _tc2_body,
        grid=(NPAD // BM,),
        in_specs=[
            pl.BlockSpec((BM, F), lambda i: (i, 0)),
            pl.BlockSpec((BM, F), lambda i: (i, 0)),
            pl.BlockSpec((BM, F), lambda i: (i, 0)),
            pl.BlockSpec((BM, 1), lambda i: (i, 0)),
            pl.BlockSpec((1, F), lambda i: (0, 0)),
            pl.BlockSpec((F, F), lambda i: (0, 0)),
        ],
        out_specs=pl.BlockSpec((BM, F), lambda i: (i, 0)),
        out_shape=jax.ShapeDtypeStruct((NPAD, F), jnp.float32),
    )(y0, y1, z1, dis, b1, w2)


def _tc3_body(y0_ref, y1_ref, z2_ref, dis_ref, b2_ref, out_ref):
    o = dis_ref[...] * (y0_ref[...] + y1_ref[...] + z2_ref[...]) + b2_ref[...]
    out_ref[...] = jnp.maximum(o, 0.0)


def _tc3(y0, y1, z2, dis, b2):
    return pl.pallas_call(
        _tc3_body,
        grid=(NPAD // BM,),
        in_specs=[
            pl.BlockSpec((BM, F), lambda i: (i, 0)),
            pl.BlockSpec((BM, F), lambda i: (i, 0)),
            pl.BlockSpec((BM, F), lambda i: (i, 0)),
            pl.BlockSpec((BM, 1), lambda i: (i, 0)),
            pl.BlockSpec((1, F), lambda i: (0, 0)),
        ],
        out_specs=pl.BlockSpec((BM, F), lambda i: (i, 0)),
        out_shape=jax.ShapeDtypeStruct((NPAD, F), jnp.float32),
    )(y0, y1, z2, dis, b2)


@jax.jit
def kernel(x, edge_index, W1, b1, W2, b2):
    src = edge_index[0].astype(jnp.int32)
    dst = edge_index[1].astype(jnp.int32)
    pad_src = jnp.arange(EPAD - E, dtype=jnp.int32) % N
    pad_dst = jnp.full((EPAD - E,), N, dtype=jnp.int32)
    src_p = jnp.concatenate([src, pad_src]).reshape(NW, NCHUNK, CHUNK)
    dst_p = jnp.concatenate([dst, pad_dst]).reshape(NW, NCHUNK, CHUNK)
    x_p = jnp.pad(x, ((0, NPAD - N), (0, 0)))

    zeros_n = jnp.zeros((NPAD,), jnp.float32)
    ones_chunk = jnp.ones((CHUNK,), jnp.float32)
    zeros_rows = jnp.zeros((NPAD, F), jnp.float32)

    deg = _sc_degree(dst_p, zeros_n, ones_chunk)
    d0 = deg[0].reshape(NPAD, 1)
    d1 = deg[1].reshape(NPAD, 1)

    z1, dis = _tc1(x_p, d0, d1, W1)
    y1 = _sc_scatter(src_p, dst_p, z1, zeros_rows)
    z2 = _tc2(y1[0], y1[1], z1, dis, b1.reshape(1, F), W2)
    y2 = _sc_scatter(src_p, dst_p, z2, zeros_rows)
    out = _tc3(y2[0], y2[1], z2, dis, b2.reshape(1, F))
    return out[:N]


# speedup vs baseline: 1.0054x; 1.0008x over previous
import functools

import jax
import jax.numpy as jnp
from jax import lax
from jax.experimental import pallas as pl
from jax.experimental.pallas import tpu as pltpu
from jax.experimental.pallas import tpu_sc as plsc

N = 10000
F = 128
E = 320000

NPAD = 10240
NC = 2
NS = 16
NW = NC * NS
CHUNK = 128
EPAD = 327680
EPW = EPAD // NW
NCHUNK = EPW // CHUNK
ROWS_PER_TILE = NPAD // NS

_mesh = plsc.VectorSubcoreMesh(core_axis_name="c", subcore_axis_name="s")


@functools.partial(
    pl.kernel,
    out_type=jax.ShapeDtypeStruct((NC, NPAD), jnp.float32),
    mesh=_mesh,
    scratch_types=[
        pltpu.VMEM((NCHUNK, CHUNK), jnp.int32),
        pltpu.VMEM((CHUNK,), jnp.float32),
        pltpu.VMEM_SHARED((NPAD,), jnp.float32),
    ],
)
def _sc_degree(dst_hbm, zeros_hbm, ones_hbm, out_hbm, idx_v, ones_v, deg_sh):
    cid = lax.axis_index("c")
    sid = lax.axis_index("s")
    wid = sid * NC + cid
    row0 = sid * ROWS_PER_TILE
    pltpu.sync_copy(dst_hbm.at[wid], idx_v)
    pltpu.sync_copy(zeros_hbm.at[pl.ds(row0, ROWS_PER_TILE)],
                    deg_sh.at[pl.ds(row0, ROWS_PER_TILE)])
    pltpu.sync_copy(ones_hbm, ones_v)
    plsc.subcore_barrier()

    def body(j, carry):
        pltpu.sync_copy(ones_v, deg_sh.at[idx_v.at[j]], add=True)
        return carry

    lax.fori_loop(0, NCHUNK, body, 0)
    plsc.subcore_barrier()
    pltpu.sync_copy(
        deg_sh.at[pl.ds(row0, ROWS_PER_TILE)],
        out_hbm.at[cid, pl.ds(row0, ROWS_PER_TILE)],
    )


@functools.partial(
    pl.kernel,
    out_type=jax.ShapeDtypeStruct((NC, NPAD, F), jnp.float32),
    mesh=_mesh,
    scratch_types=[
        pltpu.VMEM((NCHUNK, CHUNK), jnp.int32),
        pltpu.VMEM((2, CHUNK), jnp.int32),
        pltpu.VMEM((2, CHUNK, F), jnp.float32),
        pltpu.VMEM_SHARED((NPAD, F), jnp.float32),
        pltpu.SemaphoreType.DMA((2,)),
        pltpu.SemaphoreType.DMA((2,)),
    ],
)
def _sc_scatter(src_hbm, dst_hbm, z_hbm, zrows_hbm, out_hbm,
                src_v, dst_v, rows_v, y_sh, sems, semd):
    cid = lax.axis_index("c")
    sid = lax.axis_index("s")
    wid = sid * NC + cid
    row0 = sid * ROWS_PER_TILE
    pltpu.sync_copy(src_hbm.at[wid], src_v)
    pltpu.sync_copy(zrows_hbm.at[pl.ds(row0, ROWS_PER_TILE)],
                    y_sh.at[pl.ds(row0, ROWS_PER_TILE)])
    plsc.subcore_barrier()
    pltpu.async_copy(z_hbm.at[src_v.at[0]], rows_v.at[0], sems.at[0])
    pltpu.async_copy(dst_hbm.at[wid, 0], dst_v.at[0], semd.at[0])

    def body(j, carry):
        slot = lax.rem(j, 2)
        nslot = lax.rem(j + 1, 2)

        @pl.when(j + 1 < NCHUNK)
        def _start_next():
            pltpu.async_copy(
                z_hbm.at[src_v.at[j + 1]], rows_v.at[nslot], sems.at[nslot])
            pltpu.async_copy(
                dst_hbm.at[wid, j + 1], dst_v.at[nslot], semd.at[nslot])

        pltpu.make_async_copy(
            z_hbm.at[src_v.at[j]], rows_v.at[slot], sems.at[slot]).wait()
        pltpu.make_async_copy(
            dst_hbm.at[wid, j], dst_v.at[slot], semd.at[slot]).wait()
        pltpu.sync_copy(rows_v.at[slot], y_sh.at[dst_v.at[slot]], add=True)
        return carry

    lax.fori_loop(0, NCHUNK, body, 0)
    plsc.subcore_barrier()
    pltpu.sync_copy(
        y_sh.at[pl.ds(row0, ROWS_PER_TILE)],
        out_hbm.at[cid, pl.ds(row0, ROWS_PER_TILE)],
    )


BM = 2048


def _tc0_body(x_ref, w_ref, xw_ref):
    xw_ref[...] = jnp.dot(
        x_ref[...], w_ref[...], preferred_element_type=jnp.float32)


def _tc0(x, w1):
    return pl.pallas_call(
        _tc0_body,
        grid=(NPAD // BM,),
        in_specs=[
            pl.BlockSpec((BM, F), lambda i: (i, 0)),
            pl.BlockSpec((F, F), lambda i: (0, 0)),
        ],
        out_specs=pl.BlockSpec((BM, F), lambda i: (i, 0)),
        out_shape=jax.ShapeDtypeStruct((NPAD, F), jnp.float32),
    )(x, w1)


def _tc1_body(xw_ref, d0_ref, d1_ref, z_ref, dis_ref):
    deg = d0_ref[...] + d1_ref[...] + 1.0
    dis = lax.rsqrt(deg)
    z_ref[...] = xw_ref[...] * dis
    dis_ref[...] = dis


def _tc1(xw, d0, d1):
    return pl.pallas_call(
        _tc1_body,
        grid=(NPAD // BM,),
        in_specs=[
            pl.BlockSpec((BM, F), lambda i: (i, 0)),
            pl.BlockSpec((BM, 1), lambda i: (i, 0)),
            pl.BlockSpec((BM, 1), lambda i: (i, 0)),
        ],
        out_specs=[
            pl.BlockSpec((BM, F), lambda i: (i, 0)),
            pl.BlockSpec((BM, 1), lambda i: (i, 0)),
        ],
        out_shape=[
            jax.ShapeDtypeStruct((NPAD, F), jnp.float32),
            jax.ShapeDtypeStruct((NPAD, 1), jnp.float32),
        ],
    )(xw, d0, d1)


def _tc2_body(y0_ref, y1_ref, z1_ref, dis_ref, b1_ref, w2_ref, z2_ref):
    dis = dis_ref[...]
    h = dis * (y0_ref[...] + y1_ref[...] + z1_ref[...]) + b1_ref[...]
    h = jnp.maximum(h, 0.0)
    z2_ref[...] = jnp.dot(h, w2_ref[...], preferred_element_type=jnp.float32) * dis


def _tc2(y0, y1, z1, dis, b1, w2):
    return pl.pallas_call(
        _tc2_body,
        grid=(NPAD // BM,),
        in_specs=[
            pl.BlockSpec((BM, F), lambda i: (i, 0)),
            pl.BlockSpec((BM, F), lambda i: (i, 0)),
            pl.BlockSpec((BM, F), lambda i: (i, 0)),
            pl.BlockSpec((BM, 1), lambda i: (i, 0)),
            pl.BlockSpec((1, F), lambda i: (0, 0)),
            pl.BlockSpec((F, F), lambda i: (0, 0)),
        ],
        out_specs=pl.BlockSpec((BM, F), lambda i: (i, 0)),
        out_shape=jax.ShapeDtypeStruct((NPAD, F), jnp.float32),
    )(y0, y1, z1, dis, b1, w2)


def _tc3_body(y0_ref, y1_ref, z2_ref, dis_ref, b2_ref, out_ref):
    o = dis_ref[...] * (y0_ref[...] + y1_ref[...] + z2_ref[...]) + b2_ref[...]
    out_ref[...] = jnp.maximum(o, 0.0)


def _tc3(y0, y1, z2, dis, b2):
    return pl.pallas_call(
        _tc3_body,
        grid=(NPAD // BM,),
        in_specs=[
            pl.BlockSpec((BM, F), lambda i: (i, 0)),
            pl.BlockSpec((BM, F), lambda i: (i, 0)),
            pl.BlockSpec((BM, F), lambda i: (i, 0)),
            pl.BlockSpec((BM, 1), lambda i: (i, 0)),
            pl.BlockSpec((1, F), lambda i: (0, 0)),
        ],
        out_specs=pl.BlockSpec((BM, F), lambda i: (i, 0)),
        out_shape=jax.ShapeDtypeStruct((NPAD, F), jnp.float32),
    )(y0, y1, z2, dis, b2)


@jax.jit
def kernel(x, edge_index, W1, b1, W2, b2):
    src = edge_index[0].astype(jnp.int32)
    dst = edge_index[1].astype(jnp.int32)
    pad_src = jnp.arange(EPAD - E, dtype=jnp.int32) % N
    pad_dst = jnp.full((EPAD - E,), N, dtype=jnp.int32)
    src_p = jnp.concatenate([src, pad_src]).reshape(NW, NCHUNK, CHUNK)
    dst_p = jnp.concatenate([dst, pad_dst]).reshape(NW, NCHUNK, CHUNK)
    x_p = jnp.pad(x, ((0, NPAD - N), (0, 0)))

    zeros_n = jnp.zeros((NPAD,), jnp.float32)
    ones_chunk = jnp.ones((CHUNK,), jnp.float32)
    zeros_rows = jnp.zeros((NPAD, F), jnp.float32)

    xw = _tc0(x_p, W1)
    deg = _sc_degree(dst_p, zeros_n, ones_chunk)
    d0 = deg[0].reshape(NPAD, 1)
    d1 = deg[1].reshape(NPAD, 1)

    z1, dis = _tc1(xw, d0, d1)
    y1 = _sc_scatter(src_p, dst_p, z1, zeros_rows)
    z2 = _tc2(y1[0], y1[1], z1, dis, b1.reshape(1, F), W2)
    y2 = _sc_scatter(src_p, dst_p, z2, zeros_rows)
    out = _tc3(y2[0], y2[1], z2, dis, b2.reshape(1, F))
    return out[:N]
